# baseline (speedup 1.0000x reference)
import jax
import jax.numpy as jnp
from jax import lax
from jax.experimental import pallas as pl
from jax.experimental.pallas import tpu as pltpu
from jax.experimental.pallas import tpu_sc as plsc

B = 32
C = 96
H = 96
W = 96
HP = H + 2
WP = W + 2
RPAD = 104
PITCH = 128
IN_IMG = H * W
OUT_IMG = RPAD * PITCH
CH_PER = 2
NSTEP = C // CH_PER
IN_CHUNK = CH_PER * IN_IMG
OUT_CHUNK = CH_PER * OUT_IMG

NC = 2
NS = 16


def _body(x_hbm, out_hbm, in_v, out_v, pad_v):
    b = lax.axis_index("s") * NC + lax.axis_index("c")

    third = jnp.float32(1.0 / 3.0)

    one = jnp.float32(1.0)
    zero = jnp.float32(0.0)
    pb = b % 16
    tz = jnp.where(pb < 4, zero, one)
    bz = jnp.where(pb >= 12, zero, one)
    lz = jnp.where(b % 4 == 0, zero, one)
    rz = jnp.where(b % 4 == 3, zero, one)

    pad_v[pl.ds(W, 16)] = jnp.zeros((16,), jnp.float32)

    def step(t, carry):
        in_off = b * (C * IN_IMG) + t * IN_CHUNK
        out_off = b * (C * OUT_IMG) + t * OUT_CHUNK
        pltpu.sync_copy(x_hbm.at[pl.ds(in_off, IN_CHUNK)], in_v)

        for img in range(CH_PER):
            ib = img * IN_IMG
            ob = img * OUT_IMG

            for srow, dbase, mz in (
                (0, ob, tz),
                ((H - 1) * W, ob + (HP - 1) * PITCH, bz),
            ):
                for k in range(W // 16):
                    pad_v[pl.ds(k * 16, 16)] = in_v[pl.ds(ib + srow + k * 16, 16)]
                out_v[pl.ds(dbase + 82, 16)] = pad_v[pl.ds(W - 16, 16)] * (mz * rz)
                out_v[pl.ds(dbase, 16)] = pad_v[pl.ds(0, 16)] * (mz * lz)
                for k in range(W // 16):
                    j = k * 16
                    tv = (pad_v[pl.ds(j, 16)] + pad_v[pl.ds(j + 1, 16)] +
                          pad_v[pl.ds(j + 2, 16)]) * third * mz
                    out_v[pl.ds(dbase + j + 1, 16)] = tv

            def row(r, c2):
                src = ib + r * W
                dst = ob + (r + 1) * PITCH
                ra = in_v[pl.ds(src + W - 18, 16)]
                rb = in_v[pl.ds(src + W - 17, 16)]
                rc = in_v[pl.ds(src + W - 16, 16)]
                out_v[pl.ds(dst + 82, 16)] = (ra + rb + rc) * third * rz
                la = in_v[pl.ds(src, 16)]
                lb = in_v[pl.ds(src + 1, 16)]
                lc = in_v[pl.ds(src + 2, 16)]
                out_v[pl.ds(dst, 16)] = (la + lb + lc) * third * lz
                out_v[pl.ds(dst + 1, 16)] = la
                for k in range(1, W // 16):
                    out_v[pl.ds(dst + 1 + k * 16, 16)] = \
                        in_v[pl.ds(src + k * 16, 16)]
                return c2

            lax.fori_loop(0, H, row, 0, unroll=4)

        pltpu.sync_copy(out_v, out_hbm.at[pl.ds(out_off, OUT_CHUNK)])
        return carry

    lax.fori_loop(0, NSTEP, step, 0)


@jax.jit
def kernel(x):
    mesh = plsc.VectorSubcoreMesh(
        core_axis_name="c", subcore_axis_name="s",
        num_cores=NC, num_subcores=NS,
    )
    run = pl.kernel(
        _body,
        out_type=jax.ShapeDtypeStruct((B * C * OUT_IMG,), jnp.float32),
        mesh=mesh,
        compiler_params=pltpu.CompilerParams(needs_layout_passes=False),
        scratch_types=[
            pltpu.VMEM((IN_CHUNK,), jnp.float32),
            pltpu.VMEM((OUT_CHUNK,), jnp.float32),
            pltpu.VMEM((112,), jnp.float32),
        ],
    )
    y = run(x.reshape(-1))
    return y.reshape(B, C, RPAD, PITCH)[:, :, :HP, :WP]
